# baseline (speedup 1.0000x reference)
import functools

import jax
import jax.numpy as jnp
from jax import lax
from jax.experimental import pallas as pl
from jax.experimental.pallas import tpu as pltpu
from jax.experimental.pallas import tpu_sc as plsc

_NC = 2
_NS = 16
_NW = _NC * _NS
_LANES = 16


def _make_sc_kernel(B, S, D):
    sc_batch = B - 1
    rows_per_w = S // _NW
    CH = 32
    n_chunks = rows_per_w // CH
    vregs_per_row = D // _LANES

    mesh = plsc.VectorSubcoreMesh(core_axis_name="c", subcore_axis_name="s")

    @functools.partial(
        pl.kernel,
        out_type=jax.ShapeDtypeStruct((1, S, D), jnp.float32),
        mesh=mesh,
        compiler_params=pltpu.CompilerParams(use_tc_tiling_on_sc=True),
        scratch_types=[
            pltpu.VMEM((2, CH, D), jnp.float32),
            pltpu.VMEM((2, CH, D), jnp.float32),
            pltpu.VMEM((2, CH, D), jnp.float32),
            pltpu.SemaphoreType.DMA((2,)),
            pltpu.SemaphoreType.DMA((2,)),
        ],
    )
    def sc_kernel(in_hbm, pos_hbm, out_hbm, in_b, pos_b, out_b, sin, sout):
        wid = lax.axis_index("s") * _NC + lax.axis_index("c")
        row_base = wid * rows_per_w

        def in_descs(k, t):
            r0 = row_base + k * CH
            return [
                pltpu.make_async_copy(pos_hbm.at[pl.ds(r0, CH), :], pos_b.at[t], sin.at[t]),
                pltpu.make_async_copy(
                    in_hbm.at[sc_batch, pl.ds(r0, CH), :], in_b.at[t], sin.at[t]
                ),
            ]

        def out_descs(k, t):
            r0 = row_base + k * CH
            return [
                pltpu.make_async_copy(
                    out_b.at[t], out_hbm.at[0, pl.ds(r0, CH), :], sout.at[t]
                )
            ]

        def start_in(k, t):
            for d in in_descs(k, t):
                d.start()

        def compute(t):
            @plsc.parallel_loop(0, CH)
            def _(row):
                for c in range(vregs_per_row):
                    cs = pl.ds(c * _LANES, _LANES)
                    out_b[t, row, cs] = in_b[t, row, cs] + pos_b[t, row, cs]

        start_in(0, 0)
        start_in(1, 1)

        @pl.loop(0, n_chunks)
        def _(k):
            t = lax.rem(k, 2)
            for d in in_descs(k, t):
                d.wait()

            @pl.when(k >= 2)
            def _():
                for d in out_descs(k - 2, t):
                    d.wait()

            compute(t)
            for d in out_descs(k, t):
                d.start()

            @pl.when(k + 2 < n_chunks)
            def _():
                start_in(k + 2, t)

        for k in (n_chunks - 2, n_chunks - 1):
            for d in out_descs(k, k % 2):
                d.wait()

    return sc_kernel


def _tc_add_body(x_ref, p_ref, o_ref):
    o_ref[...] = x_ref[...] + p_ref[...]


def _tc_part(input_embeddings, pos, n_batch):
    B, S, D = input_embeddings.shape
    return pl.pallas_call(
        _tc_add_body,
        grid=(n_batch,),
        in_specs=[
            pl.BlockSpec((1, S, D), lambda b: (b, 0, 0)),
            pl.BlockSpec((S, D), lambda b: (0, 0)),
        ],
        out_specs=pl.BlockSpec((1, S, D), lambda b: (b, 0, 0)),
        out_shape=jax.ShapeDtypeStruct((n_batch, S, D), jnp.float32),
    )(input_embeddings, pos)


def kernel(input_embeddings, pos_table):
    B, S, D = input_embeddings.shape
    pos = pos_table[:S]
    sc_out = _make_sc_kernel(B, S, D)(input_embeddings, pos)
    tc_out = _tc_part(input_embeddings, pos, B - 1)
    return jnp.concatenate([tc_out, sc_out], axis=0)


# speedup vs baseline: 1.2627x; 1.2627x over previous
import functools

import jax
import jax.numpy as jnp
from jax import lax
from jax.experimental import pallas as pl
from jax.experimental.pallas import tpu as pltpu
from jax.experimental.pallas import tpu_sc as plsc

_NC = 2
_NS = 16
_NW = _NC * _NS
_LANES = 16
_NSLOT = 4


def _make_sc_kernel(B, S, D):
    rows_per_w = S // _NW
    CH = 16
    n_chunks = rows_per_w // CH
    vregs_per_row = D // _LANES

    mesh = plsc.VectorSubcoreMesh(core_axis_name="c", subcore_axis_name="s")

    @functools.partial(
        pl.kernel,
        out_type=jax.ShapeDtypeStruct((B, S, D), jnp.float32),
        mesh=mesh,
        compiler_params=pltpu.CompilerParams(use_tc_tiling_on_sc=True),
        scratch_types=[
            pltpu.VMEM((_NSLOT, B, CH, D), jnp.float32),
            pltpu.VMEM((_NSLOT, CH, D), jnp.float32),
            pltpu.SemaphoreType.DMA((_NSLOT,)),
            pltpu.SemaphoreType.DMA((_NSLOT,)),
        ],
    )
    def sc_kernel(in_hbm, pos_hbm, out_hbm, io_b, pos_b, sin, sout):
        wid = lax.axis_index("s") * _NC + lax.axis_index("c")
        row_base = wid * rows_per_w

        def in_descs(k, t):
            r0 = row_base + k * CH
            descs = [
                pltpu.make_async_copy(pos_hbm.at[pl.ds(r0, CH), :], pos_b.at[t], sin.at[t])
            ]
            for b in range(B):
                descs.append(
                    pltpu.make_async_copy(
                        in_hbm.at[b, pl.ds(r0, CH), :], io_b.at[t, b], sin.at[t]
                    )
                )
            return descs

        def out_descs(k, t):
            r0 = row_base + k * CH
            return [
                pltpu.make_async_copy(
                    io_b.at[t, b], out_hbm.at[b, pl.ds(r0, CH), :], sout.at[t]
                )
                for b in range(B)
            ]

        def start_in(k, t):
            for d in in_descs(k, t):
                d.start()

        def compute(t):
            @plsc.parallel_loop(0, CH)
            def _(row):
                for c in range(vregs_per_row):
                    cs = pl.ds(c * _LANES, _LANES)
                    po = pos_b[t, row, cs]
                    for b in range(B):
                        plsc.addupdate(io_b.at[t, b, row, cs], po)

        start_in(0, 0)
        start_in(1, 1)

        @pl.loop(0, n_chunks)
        def _(k):
            t = lax.rem(k, _NSLOT)
            for d in in_descs(k, t):
                d.wait()
            compute(t)
            for d in out_descs(k, t):
                d.start()

            @pl.when(k >= 2)
            def _():
                for d in out_descs(k - 2, lax.rem(k - 2, _NSLOT)):
                    d.wait()

            @pl.when(k + 2 < n_chunks)
            def _():
                start_in(k + 2, lax.rem(k + 2, _NSLOT))

        for k in (n_chunks - 2, n_chunks - 1):
            for d in out_descs(k, k % _NSLOT):
                d.wait()

    return sc_kernel


def kernel(input_embeddings, pos_table):
    B, S, D = input_embeddings.shape
    return _make_sc_kernel(B, S, D)(input_embeddings, pos_table[:S])
